# baseline (speedup 1.0000x reference)
import numpy as np
import jax
from jax import lax
import jax.numpy as jnp
from jax.experimental import pallas as pl
from jax.experimental.pallas import tpu as pltpu
from jax.experimental.pallas import tpu_sc as plsc

CH = 32
G = 32
NT = G * G
A = -0.75
NWORKERS = 32
CHUNK = 512
GRP = 16


def _cubic(t):
    t2 = t * t
    t3 = t2 * t
    w0 = A * (t3 - 2.0 * t2 + t)
    w1 = (A + 2.0) * t3 - (A + 3.0) * t2 + 1.0
    u = 1.0 - t
    u2 = u * u
    u3 = u2 * u
    w2 = (A + 2.0) * u3 - (A + 3.0) * u2 + 1.0
    w3 = A * (u3 - 2.0 * u2 + u)
    return (w0, w1, w2, w3)


def _axis_taps(v):
    s = v * (0.5 * (G - 1)) + (0.5 * (G - 1))
    i0 = s.astype(jnp.int32)
    t = s - i0.astype(jnp.float32)
    ws = _cubic(t)
    idx = tuple(jnp.clip(i0 + k, 0, G - 1) for k in (-1, 0, 1, 2))
    return idx, ws


def _sc_body(xs_hbm, ys_hbm, zs_hbm, tab_hbm, embt_hbm,
             xv, yv, zv, tab_v, out_v):
    npw = xs_hbm.shape[0] // NWORKERS
    wid = lax.axis_index("s") * 2 + lax.axis_index("c")
    base = wid * npw
    pltpu.sync_copy(tab_hbm, tab_v)

    def chunk_body(ci, carry):
        off = base + ci * CHUNK
        pltpu.sync_copy(xs_hbm.at[pl.ds(off, CHUNK)], xv)
        pltpu.sync_copy(ys_hbm.at[pl.ds(off, CHUNK)], yv)
        pltpu.sync_copy(zs_hbm.at[pl.ds(off, CHUNK)], zv)

        def group_body(g, c2):
            xx = xv[pl.ds(g * GRP, GRP)]
            yy = yv[pl.ds(g * GRP, GRP)]
            zz = zv[pl.ds(g * GRP, GRP)]
            xi, xw = _axis_taps(xx)
            yi, yw = _axis_taps(yy)
            zi, zw = _axis_taps(zz)
            accs = [jnp.zeros((GRP,), jnp.float32) for _ in range(CH)]
            planes = ((0, yi, yw, xi, xw),
                      (1, zi, zw, yi, yw),
                      (2, zi, zw, xi, xw))
            for p, ri, rw, ci_, cw in planes:
                poff = p * (NT * CH)
                for j in range(4):
                    rowb = poff + ri[j] * (G * CH)
                    for i in range(4):
                        bb = rowb + ci_[i] * CH
                        w = rw[j] * cw[i]
                        for ch in range(CH):
                            gv = plsc.load_gather(tab_v, [bb + ch])
                            accs[ch] = accs[ch] + w * gv
            for ch in range(CH):
                out_v[ch, pl.ds(g * GRP, GRP)] = accs[ch]
            return c2

        lax.fori_loop(0, CHUNK // GRP, group_body, 0, unroll=False)
        pltpu.sync_copy(out_v, embt_hbm.at[:, pl.ds(off, CHUNK)])
        return carry

    lax.fori_loop(0, npw // CHUNK, chunk_body, 0, unroll=False)


def _tail_body(embt_ref, bf_ref, o_ref):
    e = embt_ref[...]
    emb = e.T
    proj = jnp.dot(emb, bf_ref[...], preferred_element_type=jnp.float32)
    proj = proj * (2.0 * np.pi)
    o_ref[...] = jnp.concatenate([jnp.sin(proj), jnp.cos(proj)], axis=1)


def kernel(coordinates, plane4_x, plane4_y, plane4_z, B_fourier,
           iteration=0, is_training=0):
    N = coordinates.shape[0]
    ct = coordinates.T
    xs, ys, zs = ct[0], ct[1], ct[2]
    tab = jnp.concatenate(
        [jnp.transpose(p, (1, 2, 0)).reshape(-1)
         for p in (plane4_x, plane4_y, plane4_z)], axis=0)

    embt = pl.kernel(
        _sc_body,
        out_type=jax.ShapeDtypeStruct((CH, N), jnp.float32),
        mesh=plsc.VectorSubcoreMesh(core_axis_name="c", subcore_axis_name="s"),
        compiler_params=pltpu.CompilerParams(needs_layout_passes=False),
        scratch_types=[
            pltpu.VMEM((CHUNK,), jnp.float32),
            pltpu.VMEM((CHUNK,), jnp.float32),
            pltpu.VMEM((CHUNK,), jnp.float32),
            pltpu.VMEM((3 * NT * CH,), jnp.float32),
            pltpu.VMEM((CH, CHUNK), jnp.float32),
        ],
    )(xs, ys, zs, tab)

    B = 2048
    out = pl.pallas_call(
        _tail_body,
        grid=(N // B,),
        in_specs=[
            pl.BlockSpec((CH, B), lambda i: (0, i)),
            pl.BlockSpec((CH, CH // 2), lambda i: (0, 0)),
        ],
        out_specs=pl.BlockSpec((B, CH), lambda i: (i, 0)),
        out_shape=jax.ShapeDtypeStruct((N, CH), jnp.float32),
    )(embt, B_fourier)
    return out


# speedup vs baseline: 3.2448x; 3.2448x over previous
import numpy as np
import jax
from jax import lax
import jax.numpy as jnp
from jax.experimental import pallas as pl
from jax.experimental.pallas import tpu as pltpu
from jax.experimental.pallas import tpu_sc as plsc

CH = 32
CW = CH // 2
G = 32
NT = G * G
A = -0.75
NWORKERS = 32
CHUNK = 512
GRP = 16


def _cubic(t):
    t2 = t * t
    t3 = t2 * t
    w0 = A * (t3 - 2.0 * t2 + t)
    w1 = (A + 2.0) * t3 - (A + 3.0) * t2 + 1.0
    u = 1.0 - t
    u2 = u * u
    u3 = u2 * u
    w2 = (A + 2.0) * u3 - (A + 3.0) * u2 + 1.0
    w3 = A * (u3 - 2.0 * u2 + u)
    return (w0, w1, w2, w3)


def _axis_taps(v):
    s = v * (0.5 * (G - 1)) + (0.5 * (G - 1))
    i0 = s.astype(jnp.int32)
    t = s - i0.astype(jnp.float32)
    ws = _cubic(t)
    idx = tuple(jnp.clip(i0 + k, 0, G - 1) for k in (-1, 0, 1, 2))
    return idx, ws


def _sc_body(xs_hbm, ys_hbm, zs_hbm, tab_hbm, embt_hbm,
             xv, yv, zv, tab_v, out_v):
    npw = xs_hbm.shape[0] // NWORKERS
    wid = lax.axis_index("s") * 2 + lax.axis_index("c")
    base = wid * npw
    pltpu.sync_copy(tab_hbm, tab_v)

    def chunk_body(ci, carry):
        off = base + ci * CHUNK
        pltpu.sync_copy(xs_hbm.at[pl.ds(off, CHUNK)], xv)
        pltpu.sync_copy(ys_hbm.at[pl.ds(off, CHUNK)], yv)
        pltpu.sync_copy(zs_hbm.at[pl.ds(off, CHUNK)], zv)

        def group_body(g, c2):
            xx = xv[pl.ds(g * GRP, GRP)]
            yy = yv[pl.ds(g * GRP, GRP)]
            zz = zv[pl.ds(g * GRP, GRP)]
            xi, xw = _axis_taps(xx)
            yi, yw = _axis_taps(yy)
            zi, zw = _axis_taps(zz)
            accs = [jnp.zeros((2 * GRP,), jnp.bfloat16) for _ in range(CW)]
            planes = ((0, yi, yw, xi, xw),
                      (1, zi, zw, yi, yw),
                      (2, zi, zw, xi, xw))
            for p, ri, rw, ci_, cw in planes:
                poff = p * (NT * CW)
                for j in range(4):
                    rowb = poff + ri[j] * (G * CW)
                    for i in range(4):
                        bb = rowb + ci_[i] * CW
                        w = rw[j] * cw[i]
                        wp = plsc.pack(w, w, format=plsc.PackFormat.INTERLEAVED)
                        for chw in range(CW):
                            gv = plsc.load_gather(tab_v, [bb + chw])
                            gb = plsc.bitcast(gv, jnp.bfloat16)
                            accs[chw] = accs[chw] + wp * gb
            for chw in range(CW):
                a, b = plsc.unpack(accs[chw],
                                   format=plsc.PackFormat.INTERLEAVED,
                                   preferred_element_type=jnp.float32)
                out_v[2 * chw, pl.ds(g * GRP, GRP)] = a
                out_v[2 * chw + 1, pl.ds(g * GRP, GRP)] = b
            return c2

        lax.fori_loop(0, CHUNK // GRP, group_body, 0, unroll=False)
        pltpu.sync_copy(out_v, embt_hbm.at[:, pl.ds(off, CHUNK)])
        return carry

    lax.fori_loop(0, npw // CHUNK, chunk_body, 0, unroll=False)


def _tail_body(embt_ref, bf_ref, o_ref):
    e = embt_ref[...]
    emb = e.T
    proj = jnp.dot(emb, bf_ref[...], preferred_element_type=jnp.float32)
    proj = proj * (2.0 * np.pi)
    o_ref[...] = jnp.concatenate([jnp.sin(proj), jnp.cos(proj)], axis=1)


def kernel(coordinates, plane4_x, plane4_y, plane4_z, B_fourier,
           iteration=0, is_training=0):
    N = coordinates.shape[0]
    ct = coordinates.T
    xs, ys, zs = ct[0], ct[1], ct[2]
    tab = jnp.concatenate(
        [jnp.transpose(p, (1, 2, 0)).reshape(-1)
         for p in (plane4_x, plane4_y, plane4_z)], axis=0)
    tabw = jax.lax.bitcast_convert_type(
        tab.astype(jnp.bfloat16).reshape(-1, 2), jnp.int32)

    embt = pl.kernel(
        _sc_body,
        out_type=jax.ShapeDtypeStruct((CH, N), jnp.float32),
        mesh=plsc.VectorSubcoreMesh(core_axis_name="c", subcore_axis_name="s"),
        compiler_params=pltpu.CompilerParams(needs_layout_passes=False),
        scratch_types=[
            pltpu.VMEM((CHUNK,), jnp.float32),
            pltpu.VMEM((CHUNK,), jnp.float32),
            pltpu.VMEM((CHUNK,), jnp.float32),
            pltpu.VMEM((3 * NT * CW,), jnp.int32),
            pltpu.VMEM((CH, CHUNK), jnp.float32),
        ],
    )(xs, ys, zs, tabw)

    B = 2048
    out = pl.pallas_call(
        _tail_body,
        grid=(N // B,),
        in_specs=[
            pl.BlockSpec((CH, B), lambda i: (0, i)),
            pl.BlockSpec((CH, CH // 2), lambda i: (0, 0)),
        ],
        out_specs=pl.BlockSpec((B, CH), lambda i: (i, 0)),
        out_shape=jax.ShapeDtypeStruct((N, CH), jnp.float32),
    )(embt, B_fourier)
    return out


# speedup vs baseline: 6.3121x; 1.9453x over previous
import numpy as np
import jax
from jax import lax
import jax.numpy as jnp
from jax.experimental import pallas as pl
from jax.experimental.pallas import tpu as pltpu
from jax.experimental.pallas import tpu_sc as plsc

CH = 32
CW = CH // 2
RSTRIDE = CW + 1
G = 32
NT = G * G
A = -0.75
NWORKERS = 32
CHUNK = 512
GRP = 16


def _cubic(t):
    t2 = t * t
    t3 = t2 * t
    w0 = A * (t3 - 2.0 * t2 + t)
    w1 = (A + 2.0) * t3 - (A + 3.0) * t2 + 1.0
    u = 1.0 - t
    u2 = u * u
    u3 = u2 * u
    w2 = (A + 2.0) * u3 - (A + 3.0) * u2 + 1.0
    w3 = A * (u3 - 2.0 * u2 + u)
    return (w0, w1, w2, w3)


def _axis_taps(v):
    s = v * (0.5 * (G - 1)) + (0.5 * (G - 1))
    i0 = s.astype(jnp.int32)
    t = s - i0.astype(jnp.float32)
    ws = _cubic(t)
    idx = tuple(jnp.clip(i0 + k, 0, G - 1) for k in (-1, 0, 1, 2))
    return idx, ws


def _sc_body(xs_hbm, ys_hbm, zs_hbm, tab_hbm, embt_hbm,
             xv, yv, zv, tab_v, out_v):
    npw = xs_hbm.shape[0] // NWORKERS
    wid = lax.axis_index("s") * 2 + lax.axis_index("c")
    base = wid * npw
    pltpu.sync_copy(tab_hbm, tab_v)

    def chunk_body(ci, carry):
        off = base + ci * CHUNK
        pltpu.sync_copy(xs_hbm.at[pl.ds(off, CHUNK)], xv)
        pltpu.sync_copy(ys_hbm.at[pl.ds(off, CHUNK)], yv)
        pltpu.sync_copy(zs_hbm.at[pl.ds(off, CHUNK)], zv)

        def group_body(g, c2):
            xx = xv[pl.ds(g * GRP, GRP)]
            yy = yv[pl.ds(g * GRP, GRP)]
            zz = zv[pl.ds(g * GRP, GRP)]
            xi, xw = _axis_taps(xx)
            yi, yw = _axis_taps(yy)
            zi, zw = _axis_taps(zz)
            accs = [jnp.zeros((2 * GRP,), jnp.bfloat16) for _ in range(CW)]
            planes = ((0, yi, yw, xi, xw),
                      (1, zi, zw, yi, yw),
                      (2, zi, zw, xi, xw))
            for p, ri, rw, ci_, cw in planes:
                poff = p * (NT * RSTRIDE)
                for j in range(4):
                    rowb = poff + ri[j] * (G * RSTRIDE)
                    for i in range(4):
                        bb = rowb + ci_[i] * RSTRIDE
                        w = rw[j] * cw[i]
                        wp = plsc.pack(w, w, format=plsc.PackFormat.INTERLEAVED)
                        for chw in range(CW):
                            gv = plsc.load_gather(tab_v, [bb + chw])
                            gb = plsc.bitcast(gv, jnp.bfloat16)
                            accs[chw] = accs[chw] + wp * gb
            for chw in range(CW):
                a, b = plsc.unpack(accs[chw],
                                   format=plsc.PackFormat.INTERLEAVED,
                                   preferred_element_type=jnp.float32)
                out_v[2 * chw, pl.ds(g * GRP, GRP)] = a
                out_v[2 * chw + 1, pl.ds(g * GRP, GRP)] = b
            return c2

        lax.fori_loop(0, CHUNK // GRP, group_body, 0, unroll=False)
        pltpu.sync_copy(out_v, embt_hbm.at[:, pl.ds(off, CHUNK)])
        return carry

    lax.fori_loop(0, npw // CHUNK, chunk_body, 0, unroll=False)


def _tail_body(embt_ref, bf_ref, o_ref):
    e = embt_ref[...]
    emb = e.T
    proj = jnp.dot(emb, bf_ref[...], preferred_element_type=jnp.float32)
    proj = proj * (2.0 * np.pi)
    o_ref[...] = jnp.concatenate([jnp.sin(proj), jnp.cos(proj)], axis=1)


def kernel(coordinates, plane4_x, plane4_y, plane4_z, B_fourier,
           iteration=0, is_training=0):
    N = coordinates.shape[0]
    ct = coordinates.T
    xs, ys, zs = ct[0], ct[1], ct[2]
    tab = jnp.concatenate(
        [jnp.transpose(p, (1, 2, 0)).reshape(-1)
         for p in (plane4_x, plane4_y, plane4_z)], axis=0)
    tabw = jax.lax.bitcast_convert_type(
        tab.astype(jnp.bfloat16).reshape(-1, 2), jnp.int32)
    tabw = jnp.pad(tabw.reshape(3 * NT, CW), ((0, 0), (0, RSTRIDE - CW))
                   ).reshape(-1)

    embt = pl.kernel(
        _sc_body,
        out_type=jax.ShapeDtypeStruct((CH, N), jnp.float32),
        mesh=plsc.VectorSubcoreMesh(core_axis_name="c", subcore_axis_name="s"),
        compiler_params=pltpu.CompilerParams(needs_layout_passes=False),
        scratch_types=[
            pltpu.VMEM((CHUNK,), jnp.float32),
            pltpu.VMEM((CHUNK,), jnp.float32),
            pltpu.VMEM((CHUNK,), jnp.float32),
            pltpu.VMEM((3 * NT * RSTRIDE,), jnp.int32),
            pltpu.VMEM((CH, CHUNK), jnp.float32),
        ],
    )(xs, ys, zs, tabw)

    B = 2048
    out = pl.pallas_call(
        _tail_body,
        grid=(N // B,),
        in_specs=[
            pl.BlockSpec((CH, B), lambda i: (0, i)),
            pl.BlockSpec((CH, CH // 2), lambda i: (0, 0)),
        ],
        out_specs=pl.BlockSpec((B, CH), lambda i: (i, 0)),
        out_shape=jax.ShapeDtypeStruct((N, CH), jnp.float32),
    )(embt, B_fourier)
    return out


# speedup vs baseline: 6.5436x; 1.0367x over previous
import numpy as np
import jax
from jax import lax
import jax.numpy as jnp
from jax.experimental import pallas as pl
from jax.experimental.pallas import tpu as pltpu
from jax.experimental.pallas import tpu_sc as plsc

CH = 32
CW = CH // 2
RSTRIDE = CW + 1
G = 32
NT = G * G
A = -0.75
NWORKERS = 32
CHUNK = 512
GRP = 16


def _cubic(t):
    t2 = t * t
    t3 = t2 * t
    w0 = A * (t3 - 2.0 * t2 + t)
    w1 = (A + 2.0) * t3 - (A + 3.0) * t2 + 1.0
    u = 1.0 - t
    u2 = u * u
    u3 = u2 * u
    w2 = (A + 2.0) * u3 - (A + 3.0) * u2 + 1.0
    w3 = A * (u3 - 2.0 * u2 + u)
    return (w0, w1, w2, w3)


def _axis_taps(v):
    s = v * (0.5 * (G - 1)) + (0.5 * (G - 1))
    i0 = s.astype(jnp.int32)
    t = s - i0.astype(jnp.float32)
    ws = _cubic(t)
    idx = tuple(jnp.clip(i0 + k, 0, G - 1) for k in (-1, 0, 1, 2))
    return idx, ws


def _sc_body(xs_hbm, ys_hbm, zs_hbm, tab_hbm, embt_hbm,
             xv, yv, zv, tab_v, out_v, sem_in, sem_out):
    npw = xs_hbm.shape[0] // NWORKERS
    nchunks = npw // CHUNK
    wid = lax.axis_index("s") * 2 + lax.axis_index("c")
    base = wid * npw
    pltpu.sync_copy(tab_hbm, tab_v)

    def start_in(ci, slot):
        off = base + ci * CHUNK
        pltpu.make_async_copy(
            xs_hbm.at[pl.ds(off, CHUNK)], xv.at[slot], sem_in).start()
        pltpu.make_async_copy(
            ys_hbm.at[pl.ds(off, CHUNK)], yv.at[slot], sem_in).start()
        pltpu.make_async_copy(
            zs_hbm.at[pl.ds(off, CHUNK)], zv.at[slot], sem_in).start()

    def drain_in(slot):
        pltpu.make_async_copy(
            xs_hbm.at[pl.ds(0, CHUNK)], xv.at[slot], sem_in).wait()
        pltpu.make_async_copy(
            ys_hbm.at[pl.ds(0, CHUNK)], yv.at[slot], sem_in).wait()
        pltpu.make_async_copy(
            zs_hbm.at[pl.ds(0, CHUNK)], zv.at[slot], sem_in).wait()

    def drain_out(slot):
        pltpu.make_async_copy(
            out_v.at[slot], embt_hbm.at[:, pl.ds(0, CHUNK)], sem_out).wait()

    start_in(0, 0)

    def chunk_body(ci, carry):
        slot = lax.rem(ci, 2)
        off = base + ci * CHUNK

        @pl.when(ci + 1 < nchunks)
        def _():
            start_in(ci + 1, 1 - slot)

        drain_in(slot)

        @pl.when(ci >= 2)
        def _():
            drain_out(slot)

        @plsc.parallel_loop(0, CHUNK // GRP)
        def group_body(g):
            xx = xv[slot, pl.ds(g * GRP, GRP)]
            yy = yv[slot, pl.ds(g * GRP, GRP)]
            zz = zv[slot, pl.ds(g * GRP, GRP)]
            xi, xw = _axis_taps(xx)
            yi, yw = _axis_taps(yy)
            zi, zw = _axis_taps(zz)
            accs = [jnp.zeros((2 * GRP,), jnp.bfloat16) for _ in range(CW)]
            planes = ((0, yi, yw, xi, xw),
                      (1, zi, zw, yi, yw),
                      (2, zi, zw, xi, xw))
            for p, ri, rw, ci_, cw in planes:
                poff = p * (NT * RSTRIDE)
                for j in range(4):
                    rowb = poff + ri[j] * (G * RSTRIDE)
                    for i in range(4):
                        bb = rowb + ci_[i] * RSTRIDE
                        w = rw[j] * cw[i]
                        wp = plsc.pack(w, w, format=plsc.PackFormat.INTERLEAVED)
                        for chw in range(CW):
                            gv = plsc.load_gather(tab_v, [bb + chw])
                            gb = plsc.bitcast(gv, jnp.bfloat16)
                            accs[chw] = accs[chw] + wp * gb
            for chw in range(CW):
                a, b = plsc.unpack(accs[chw],
                                   format=plsc.PackFormat.INTERLEAVED,
                                   preferred_element_type=jnp.float32)
                out_v[slot, 2 * chw, pl.ds(g * GRP, GRP)] = a
                out_v[slot, 2 * chw + 1, pl.ds(g * GRP, GRP)] = b

        pltpu.make_async_copy(
            out_v.at[slot], embt_hbm.at[:, pl.ds(off, CHUNK)], sem_out).start()
        return carry

    lax.fori_loop(0, nchunks, chunk_body, 0, unroll=False)
    drain_out(lax.rem(nchunks - 2, 2))
    drain_out(lax.rem(nchunks - 1, 2))


def _tail_body(embt_ref, bf_ref, o_ref):
    e = embt_ref[...]
    emb = e.T
    proj = jnp.dot(emb, bf_ref[...], preferred_element_type=jnp.float32)
    proj = proj * (2.0 * np.pi)
    o_ref[...] = jnp.concatenate([jnp.sin(proj), jnp.cos(proj)], axis=1)


def kernel(coordinates, plane4_x, plane4_y, plane4_z, B_fourier,
           iteration=0, is_training=0):
    N = coordinates.shape[0]
    ct = coordinates.T
    xs, ys, zs = ct[0], ct[1], ct[2]
    tab = jnp.concatenate(
        [jnp.transpose(p, (1, 2, 0)).reshape(-1)
         for p in (plane4_x, plane4_y, plane4_z)], axis=0)
    tabw = jax.lax.bitcast_convert_type(
        tab.astype(jnp.bfloat16).reshape(-1, 2), jnp.int32)
    tabw = jnp.pad(tabw.reshape(3 * NT, CW), ((0, 0), (0, RSTRIDE - CW))
                   ).reshape(-1)

    embt = pl.kernel(
        _sc_body,
        out_type=jax.ShapeDtypeStruct((CH, N), jnp.float32),
        mesh=plsc.VectorSubcoreMesh(core_axis_name="c", subcore_axis_name="s"),
        compiler_params=pltpu.CompilerParams(needs_layout_passes=False),
        scratch_types=[
            pltpu.VMEM((2, CHUNK), jnp.float32),
            pltpu.VMEM((2, CHUNK), jnp.float32),
            pltpu.VMEM((2, CHUNK), jnp.float32),
            pltpu.VMEM((3 * NT * RSTRIDE,), jnp.int32),
            pltpu.VMEM((2, CH, CHUNK), jnp.float32),
            pltpu.SemaphoreType.DMA,
            pltpu.SemaphoreType.DMA,
        ],
    )(xs, ys, zs, tabw)

    B = 2048
    out = pl.pallas_call(
        _tail_body,
        grid=(N // B,),
        in_specs=[
            pl.BlockSpec((CH, B), lambda i: (0, i)),
            pl.BlockSpec((CH, CH // 2), lambda i: (0, 0)),
        ],
        out_specs=pl.BlockSpec((B, CH), lambda i: (i, 0)),
        out_shape=jax.ShapeDtypeStruct((N, CH), jnp.float32),
    )(embt, B_fourier)
    return out
